# baseline (speedup 1.0000x reference)
import functools

import jax
import jax.numpy as jnp
from jax import lax
from jax.experimental import pallas as pl
from jax.experimental.pallas import tpu as pltpu
from jax.experimental.pallas import tpu_sc as plsc

B, S = 16384, 200
V, D = 1_000_000, 32
N = B * S
NC, NS = 2, 16
NW = NC * NS
NPW = N // NW

C = 1024
NCHUNK = NPW // C
NBUF = 2
NSUPER = NCHUNK // NBUF

NL = N * D // 128
LPS = S * D // 128
WSEQ = 4
BPW = B // NW
NWIN = BPW // WSEQ

_mesh = plsc.VectorSubcoreMesh(core_axis_name="c", subcore_axis_name="s")


@functools.partial(
    pl.kernel,
    out_type=jax.ShapeDtypeStruct((N, D), jnp.float32),
    mesh=_mesh,
    scratch_types=[
        pltpu.VMEM((C,), jnp.int32),
        pltpu.VMEM((C,), jnp.int32),
        pltpu.VMEM((C, D), jnp.float32),
        pltpu.VMEM((C, D), jnp.float32),
        pltpu.SemaphoreType.DMA,
        pltpu.SemaphoreType.DMA,
        pltpu.SemaphoreType.DMA,
        pltpu.SemaphoreType.DMA,
        pltpu.SemaphoreType.DMA,
        pltpu.SemaphoreType.DMA,
    ],
    compiler_params=pltpu.CompilerParams(use_tc_tiling_on_sc=False),
)
def _gather(table_hbm, idx_hbm, out_hbm,
            idx0, idx1, rows0, rows1, is0, is1, gs0, gs1, os0, os1):
    idx_v = (idx0, idx1)
    rows_v = (rows0, rows1)
    isem = (is0, is1)
    gsem = (gs0, gs1)
    osem = (os0, os1)

    wid = lax.axis_index("s") * NC + lax.axis_index("c")
    base = wid * NPW

    def idx_src(ci):
        return idx_hbm.at[pl.ds(base + ci * C, C)]

    def out_dst(ci):
        return out_hbm.at[pl.ds(base + ci * C, C)]

    for b in range(NBUF):
        pltpu.async_copy(idx_src(b), idx_v[b], isem[b])

    def super_chunk(g, carry):
        for b in range(NBUF):
            ci = g * NBUF + b
            pltpu.make_async_copy(idx_src(ci), idx_v[b], isem[b]).wait()

            @pl.when(g > 0)
            def _():
                pltpu.make_async_copy(rows_v[b], out_dst(ci), osem[b]).wait()

            pltpu.async_copy(
                table_hbm.at[idx_v[b]], rows_v[b], gsem[b]
            ).wait()

            pltpu.async_copy(rows_v[b], out_dst(ci), osem[b])

            @pl.when(g < NSUPER - 1)
            def _():
                pltpu.async_copy(idx_src(ci + NBUF), idx_v[b], isem[b])
        return carry

    lax.fori_loop(0, NSUPER, super_chunk, 0)

    for b in range(NBUF):
        pltpu.make_async_copy(
            rows_v[b], out_dst(NCHUNK - NBUF + b), osem[b]
        ).wait()


@functools.partial(
    pl.kernel,
    out_type=jax.ShapeDtypeStruct((B, S, D), jnp.float32),
    mesh=_mesh,
    scratch_types=[
        pltpu.VMEM((WSEQ * LPS, 128), jnp.float32),
        pltpu.VMEM((WSEQ * LPS, 128), jnp.float32),
        pltpu.VMEM((1, S, D), jnp.float32),
        pltpu.VMEM((1, S, D), jnp.float32),
        pltpu.SemaphoreType.DMA,
        pltpu.SemaphoreType.DMA,
        pltpu.SemaphoreType.DMA,
        pltpu.SemaphoreType.DMA,
    ],
    compiler_params=pltpu.CompilerParams(use_tc_tiling_on_sc=True),
)
def _repack(und_hbm, out_hbm,
            in0, in1, ob0, ob1, rs0, rs1, ws0, ws1):
    in_v = (in0, in1)
    out_v = (ob0, ob1)
    rsem = (rs0, rs1)
    wsem = (ws0, ws1)

    wid = lax.axis_index("s") * NC + lax.axis_index("c")
    seq_base = wid * BPW
    lrow_base = seq_base * LPS

    def in_src(w):
        return und_hbm.at[pl.ds(lrow_base + w * WSEQ * LPS, WSEQ * LPS)]

    for b in range(2):
        pltpu.async_copy(in_src(b), in_v[b], rsem[b])

    def window(wp, carry):
        for b in range(2):
            w = wp * 2 + b
            pltpu.make_async_copy(in_src(w), in_v[b], rsem[b]).wait()

            for q in range(WSEQ):
                ob = q & 1
                seq = seq_base + w * WSEQ + q
                out_dst = out_hbm.at[pl.ds(seq, 1)]
                if q > 1:
                    pltpu.make_async_copy(out_v[ob], out_dst, wsem[ob]).wait()
                else:
                    @pl.when(w > 0)
                    def _():
                        pltpu.make_async_copy(
                            out_v[ob], out_dst, wsem[ob]
                        ).wait()

                def row(r, carry2):
                    for t in range(4):
                        j = r * 4 + t
                        for h in range(2):
                            out_v[ob][0, j, pl.ds(h * 16, 16)] = (
                                in_v[b][q * LPS + r, pl.ds(t * 32 + h * 16, 16)]
                            )
                    return carry2

                lax.fori_loop(0, LPS, row, 0)
                pltpu.async_copy(out_v[ob], out_dst, wsem[ob])

            @pl.when(w + 2 < NWIN)
            def _():
                pltpu.async_copy(in_src(w + 2), in_v[b], rsem[b])
        return carry

    lax.fori_loop(0, NWIN // 2, window, 0)

    for ob in range(2):
        pltpu.make_async_copy(
            out_v[ob],
            out_hbm.at[pl.ds(seq_base + BPW - 2 + ob, 1)],
            wsem[ob],
        ).wait()


def kernel(seqs, species, table):
    del species
    idx_flat = seqs.reshape(-1).astype(jnp.int32)
    und = _gather(table, idx_flat)
    return _repack(und.reshape(NL, 128))
